# Initial kernel scaffold; baseline (speedup 1.0000x reference)
import jax
import jax.numpy as jnp
from jax.experimental import pallas as pl


def kernel(x, edge_index, i, params):
    raise NotImplementedError("write your pallas kernel here")


# speedup vs baseline: 1.9331x; 1.9331x over previous
import functools
import math

import jax
import jax.numpy as jnp
from jax import lax
from jax.experimental import pallas as pl
from jax.experimental.pallas import tpu as pltpu
from jax.experimental.pallas import tpu_sc as plsc

N = 10000
E = 160000
D = 128
H = 64
G = 128
EPS = 1e-3

NTILE = 32
EP = 163840
PER_TILE = EP // NTILE
CH = 128
NCH = PER_TILE // CH
R = 10240
TRASH = N
RSUB = R // 16
RCH = 160
CROWS = 128
ET = 640
NT = EP // ET
F32 = jnp.float32


def _sds(shape, dtype=F32):
    return jax.ShapeDtypeStruct(shape, dtype)


def _mesh():
    return plsc.VectorSubcoreMesh(core_axis_name="c", subcore_axis_name="s")


def _sc_gather_xs_xr(x, send, recv):
    def body(x_hbm, send_hbm, recv_hbm, xs_hbm, xr_hbm,
             idx_s, idx_r, rows_s, rows_r, sem_s, sem_r):
        wid = lax.axis_index("c") * 16 + lax.axis_index("s")
        base = wid * PER_TILE

        def step(j, carry):
            off = base + j * CH
            pltpu.sync_copy(send_hbm.at[pl.ds(off, CH)], idx_s)
            pltpu.sync_copy(recv_hbm.at[pl.ds(off, CH)], idx_r)
            cs = pltpu.async_copy(x_hbm.at[idx_s], rows_s, sem_s)
            cr = pltpu.async_copy(x_hbm.at[idx_r], rows_r, sem_r)
            cs.wait()
            cr.wait()
            pltpu.sync_copy(rows_s, xs_hbm.at[pl.ds(off, CH)])
            pltpu.sync_copy(rows_r, xr_hbm.at[pl.ds(off, CH)])
            return carry

        lax.fori_loop(0, NCH, step, 0)

    k = pl.kernel(
        body,
        out_type=(_sds((EP, D)), _sds((EP, D))),
        mesh=_mesh(),
        scratch_types=[
            pltpu.VMEM((CH,), jnp.int32),
            pltpu.VMEM((CH,), jnp.int32),
            pltpu.VMEM((CH, D), F32),
            pltpu.VMEM((CH, D), F32),
            pltpu.SemaphoreType.DMA,
            pltpu.SemaphoreType.DMA,
        ],
    )
    return k(x, send, recv)


def _zero_shared(zbuf, shared, sid, width):
    def zrow(r, c2):
        def zcol(q, c3):
            zbuf[r, pl.ds(q * 16, 16)] = jnp.zeros((16,), F32)
            return c3
        return lax.fori_loop(0, width // 16, zcol, c2)

    lax.fori_loop(0, RCH, zrow, 0)
    for q in range(RSUB // RCH):
        pltpu.sync_copy(zbuf, shared.at[pl.ds(sid * RSUB + q * RCH, RCH)])


def _read_back(zbuf, shared, out_hbm, cid, sid):
    for q in range(RSUB // RCH):
        r0 = sid * RSUB + q * RCH
        pltpu.sync_copy(shared.at[pl.ds(r0, RCH)], zbuf)
        pltpu.sync_copy(zbuf, out_hbm.at[pl.ds(cid * R + r0, RCH)])


def _zero_buf(buf, nrows, width):
    def zrow(r, c2):
        def zcol(q, c3):
            buf[r, pl.ds(q * 16, 16)] = jnp.zeros((16,), F32)
            return c3
        return lax.fori_loop(0, width // 16, zcol, c2)

    lax.fori_loop(0, nrows, zrow, 0)


def _sc_scatter_edges(m, recv_m, rm_div, oh):
    def body(m_hbm, rm_hbm, rmd_hbm, oh_hbm, out_hbm, cnt_hbm,
             idx_r, idx_cr, rows, oh_rows, shared, shared_cnt):
        cid = lax.axis_index("c")
        sid = lax.axis_index("s")
        base = (cid * 16 + sid) * PER_TILE
        _zero_buf(rows, CH, D)
        for q in range(RSUB // CH):
            pltpu.sync_copy(rows, shared.at[pl.ds(sid * RSUB + q * CH, CH)])
        pltpu.sync_copy(rows.at[pl.ds(0, 8)],
                        shared_cnt.at[pl.ds(sid * 8, 8)])
        plsc.subcore_barrier()

        def step(j, carry):
            off = base + j * CH
            pltpu.sync_copy(rm_hbm.at[pl.ds(off, CH)], idx_r)
            pltpu.sync_copy(rmd_hbm.at[pl.ds(off, CH)], idx_cr)
            pltpu.sync_copy(m_hbm.at[pl.ds(off, CH)], rows)
            pltpu.sync_copy(oh_hbm.at[pl.ds(off, CH)], oh_rows)
            pltpu.sync_copy(rows, shared.at[idx_r], add=True)
            pltpu.sync_copy(oh_rows, shared_cnt.at[idx_cr], add=True)
            return carry

        lax.fori_loop(0, NCH, step, 0)
        plsc.subcore_barrier()
        for q in range(RSUB // CH):
            r0 = sid * RSUB + q * CH
            pltpu.sync_copy(shared.at[pl.ds(r0, CH)], rows)
            pltpu.sync_copy(rows, out_hbm.at[pl.ds(cid * R + r0, CH)])
        pltpu.sync_copy(shared_cnt.at[pl.ds(sid * 8, 8)],
                        rows.at[pl.ds(0, 8)])
        pltpu.sync_copy(rows.at[pl.ds(0, 8)],
                        cnt_hbm.at[pl.ds(cid * CROWS + sid * 8, 8)])

    k = pl.kernel(
        body,
        out_type=(_sds((2 * R, D)), _sds((2 * CROWS, D))),
        mesh=_mesh(),
        scratch_types=[
            pltpu.VMEM((CH,), jnp.int32),
            pltpu.VMEM((CH,), jnp.int32),
            pltpu.VMEM((CH, D), F32),
            pltpu.VMEM((CH, D), F32),
            pltpu.VMEM_SHARED((R, D), F32),
            pltpu.VMEM_SHARED((CROWS, D), F32),
        ],
    )
    return k(m, recv_m, rm_div, oh)


def _sc_gather_scatter(h, send, recv_m, width):
    def body(h_hbm, send_hbm, rm_hbm, out_hbm, idx_s, idx_r, rows, zbuf, sem,
             shared):
        cid = lax.axis_index("c")
        sid = lax.axis_index("s")
        base = (cid * 16 + sid) * PER_TILE
        _zero_shared(zbuf, shared, sid, width)
        plsc.subcore_barrier()

        def step(j, carry):
            off = base + j * CH
            pltpu.sync_copy(send_hbm.at[pl.ds(off, CH)], idx_s)
            cp = pltpu.async_copy(h_hbm.at[idx_s], rows, sem)
            pltpu.sync_copy(rm_hbm.at[pl.ds(off, CH)], idx_r)
            cp.wait()
            pltpu.sync_copy(rows, shared.at[idx_r], add=True)
            return carry

        lax.fori_loop(0, NCH, step, 0)
        plsc.subcore_barrier()
        _read_back(zbuf, shared, out_hbm, cid, sid)

    k = pl.kernel(
        body,
        out_type=_sds((2 * R, width)),
        mesh=_mesh(),
        scratch_types=[
            pltpu.VMEM((CH,), jnp.int32),
            pltpu.VMEM((CH,), jnp.int32),
            pltpu.VMEM((CH, width), F32),
            pltpu.VMEM((RCH, width), F32),
            pltpu.SemaphoreType.DMA,
            pltpu.VMEM_SHARED((R, width), F32),
        ],
    )
    return k(h, send, recv_m)


def _edge_mlp_body(xs_ref, xr_ref, recv_ref, wr_ref, ws_ref, wd_ref, wv_ref,
                   c0_ref, w2_ref, b2_ref, mext_ref, rm_ref, rmd_ref,
                   oh_ref):
    t = pl.program_id(0)
    xs = xs_ref[...]
    xr = xr_ref[...]
    diff = xr - xs
    lane = lax.broadcasted_iota(jnp.int32, (ET, D), 1)
    d2 = jnp.sum(jnp.where(lane < 3, diff * diff, 0.0), axis=1, keepdims=True)
    dists = jnp.sqrt(d2 + 1e-24)
    inv = jnp.where(d2 > 0, 1.0 / dists, 0.0)

    m1 = jnp.dot(xr, wr_ref[...], preferred_element_type=F32)
    m1 = m1 + jnp.dot(xs, ws_ref[...], preferred_element_type=F32)
    m1 = m1 + dists * wd_ref[...]
    for kdim in range(3):
        vk = jnp.sum(jnp.where(lane == kdim, diff, 0.0), axis=1,
                     keepdims=True) * inv
        m1 = m1 + vk * wv_ref[kdim, :][None, :]
    m1 = jnp.maximum(m1 + c0_ref[...], 0.0)
    m2 = jnp.dot(m1, w2_ref[...], preferred_element_type=F32) + b2_ref[...]
    m2 = jnp.maximum(m2, 0.0)

    s3 = jnp.sum(jnp.where(lane == 3, xs, 0.0), axis=1, keepdims=True)
    r3 = jnp.sum(jnp.where(lane == 3, xr, 0.0), axis=1, keepdims=True)
    rowid = t * ET + lax.broadcasted_iota(jnp.int32, (ET, 1), 0)
    maskb = (s3 <= r3) & (rowid < E)
    mext_ref[...] = m2 * maskb.astype(F32)
    rm = jnp.where(maskb, recv_ref[...], TRASH)
    rm_ref[...] = rm
    rmd_ref[...] = rm // 128
    oh_ref[...] = (lane == lax.rem(rm, 128)).astype(F32)


def _edge_mlp(xs, xr, recv2d, wr, ws, wd, wv, c0, w2, b2):
    return pl.pallas_call(
        _edge_mlp_body,
        grid=(NT,),
        in_specs=[
            pl.BlockSpec((ET, D), lambda t: (t, 0)),
            pl.BlockSpec((ET, D), lambda t: (t, 0)),
            pl.BlockSpec((ET, 1), lambda t: (t, 0)),
            pl.BlockSpec((D, 256), lambda t: (0, 0)),
            pl.BlockSpec((D, 256), lambda t: (0, 0)),
            pl.BlockSpec((1, 256), lambda t: (0, 0)),
            pl.BlockSpec((8, 256), lambda t: (0, 0)),
            pl.BlockSpec((1, 256), lambda t: (0, 0)),
            pl.BlockSpec((256, D), lambda t: (0, 0)),
            pl.BlockSpec((1, D), lambda t: (0, 0)),
        ],
        out_specs=[
            pl.BlockSpec((ET, D), lambda t: (t, 0)),
            pl.BlockSpec((ET, 1), lambda t: (t, 0)),
            pl.BlockSpec((ET, 1), lambda t: (t, 0)),
            pl.BlockSpec((ET, D), lambda t: (t, 0)),
        ],
        out_shape=[_sds((EP, D)), _sds((EP, 1), jnp.int32),
                   _sds((EP, 1), jnp.int32), _sds((EP, D))],
    )(xs, xr, recv2d, wr, ws, wd, wv, c0, w2, b2)


def _update_mlp_body(p_ref, c_ref, u1_ref, bu1_ref, u2_ref, bu2_ref,
                     h1_ref, cnt_ref):
    agg = p_ref[0] + p_ref[1]
    cnt = c_ref[0] + c_ref[1]
    cnt_ref[...] = jnp.broadcast_to(cnt, (RSUB, 8))
    a1 = jnp.maximum(
        jnp.dot(agg, u1_ref[...], preferred_element_type=F32) + bu1_ref[...],
        0.0)
    h1 = jnp.maximum(
        jnp.dot(a1, u2_ref[...], preferred_element_type=F32) + bu2_ref[...],
        0.0)
    h1_ref[...] = jnp.concatenate([h1, jnp.zeros((RSUB, D - H), F32)], axis=1)


def _update_mlp(partials, cntp, u1, bu1, u2, bu2):
    return pl.pallas_call(
        _update_mlp_body,
        grid=(R // RSUB,),
        in_specs=[
            pl.BlockSpec((2, RSUB, D), lambda t: (0, t, 0)),
            pl.BlockSpec((2, RSUB, 1), lambda t: (0, t, 0)),
            pl.BlockSpec((2 * H, 2 * H), lambda t: (0, 0)),
            pl.BlockSpec((1, 2 * H), lambda t: (0, 0)),
            pl.BlockSpec((2 * H, H), lambda t: (0, 0)),
            pl.BlockSpec((1, H), lambda t: (0, 0)),
        ],
        out_specs=[
            pl.BlockSpec((RSUB, D), lambda t: (t, 0)),
            pl.BlockSpec((RSUB, 8), lambda t: (t, 0)),
        ],
        out_shape=[_sds((R, D)), _sds((R, 8))],
    )(partials, cntp, u1, bu1, u2, bu2)


def _sage_body(h_ref, p_ref, cnt_ref, wt_ref, wb_ref, b_ref, out_ref):
    nb = p_ref[0] + p_ref[1]
    cnt = jnp.maximum(jnp.max(cnt_ref[...], axis=1, keepdims=True), 1.0)
    nb = nb / cnt
    h = jnp.dot(h_ref[...], wt_ref[...], preferred_element_type=F32)
    h = h + jnp.dot(nb, wb_ref[...], preferred_element_type=F32) + b_ref[...]
    h = jnp.maximum(h, 0.0)
    ss = jnp.sum(h * h, axis=1, keepdims=True)
    out_ref[...] = h * lax.rsqrt(jnp.maximum(ss, 1e-12))


def _sage_layer(h, partials, cnt8, wt, wb, b):
    win = wt.shape[0]
    wout = wt.shape[1]
    return pl.pallas_call(
        _sage_body,
        grid=(R // RSUB,),
        in_specs=[
            pl.BlockSpec((RSUB, win), lambda t: (t, 0)),
            pl.BlockSpec((2, RSUB, win), lambda t: (0, t, 0)),
            pl.BlockSpec((RSUB, 8), lambda t: (t, 0)),
            pl.BlockSpec((win, wout), lambda t: (0, 0)),
            pl.BlockSpec((win, wout), lambda t: (0, 0)),
            pl.BlockSpec((1, wout), lambda t: (0, 0)),
        ],
        out_specs=pl.BlockSpec((RSUB, wout), lambda t: (t, 0)),
        out_shape=_sds((R, wout)),
    )(h, partials, cnt8, wt, wb, b)


def _pool_body(offs_ref, x_ref, h_ref, z_ref):
    g = pl.program_id(0)
    start = offs_ref[g]
    end = offs_ref[g + 1]
    b0 = start // 8
    b1 = (end + 7) // 8
    neg = jnp.float32(-jnp.inf)
    pos = jnp.float32(jnp.inf)

    def step(b, carry):
        sx, sxx, mx, mn, sh, mh = carry
        xb = x_ref[pl.ds(b * 8, 8), :]
        hb = h_ref[pl.ds(b * 8, 8), :]
        row = b * 8 + lax.broadcasted_iota(jnp.int32, (8, 1), 0)
        rm = (row >= start) & (row < end)
        xv = jnp.where(rm, xb, 0.0)
        hv = jnp.where(rm, hb, 0.0)
        sx = sx + xv
        sxx = sxx + xv * xv
        mx = jnp.maximum(mx, jnp.where(rm, xb, neg))
        mn = jnp.minimum(mn, jnp.where(rm, xb, pos))
        sh = sh + hv
        mh = jnp.maximum(mh, jnp.where(rm, hb, neg))
        return (sx, sxx, mx, mn, sh, mh)

    init = (jnp.zeros((8, D), F32), jnp.zeros((8, D), F32),
            jnp.full((8, D), neg), jnp.full((8, D), pos),
            jnp.zeros((8, 4 * H), F32), jnp.full((8, 4 * H), neg))
    sx, sxx, mx, mn, sh, mh = lax.fori_loop(b0, b1, step, init)

    nf = (end - start).astype(F32)
    has = nf > 0.0
    cg = jnp.maximum(nf, 1.0)
    gsum = jnp.sum(sx, axis=0, keepdims=True)
    gss = jnp.sum(sxx, axis=0, keepdims=True)
    gmx = jnp.where(has, jnp.max(mx, axis=0, keepdims=True), 0.0)
    gmn = jnp.where(has, jnp.min(mn, axis=0, keepdims=True), 0.0)
    hsum = jnp.sum(sh, axis=0, keepdims=True)
    hmx = jnp.where(has, jnp.max(mh, axis=0, keepdims=True), 0.0)
    gmean = gsum / cg
    gvar = jnp.abs(gss / cg - gmean * gmean)
    pavg = hsum / cg
    z_ref[0] = jnp.concatenate(
        [hmx, pavg, hsum, gmean, gvar, gmx, gmn], axis=1)


def _pool(offs, x, h2):
    grid_spec = pltpu.PrefetchScalarGridSpec(
        num_scalar_prefetch=1,
        grid=(G,),
        in_specs=[
            pl.BlockSpec((N, D), lambda g, offs: (0, 0)),
            pl.BlockSpec((R, 4 * H), lambda g, offs: (0, 0)),
        ],
        out_specs=pl.BlockSpec((1, 1, 1280), lambda g, offs: (g, 0, 0)),
    )
    return pl.pallas_call(
        _pool_body,
        grid_spec=grid_spec,
        out_shape=_sds((G, 1, 1280)),
    )(offs, x, h2)


def _decoder_body(z_ref, d1_ref, db1_ref, g1_ref, bb1_ref,
                  d2_ref, db2_ref, g2_ref, bb2_ref,
                  d3_ref, db3_ref, g3_ref, bb3_ref,
                  wh_ref, bh_ref, out_ref):
    def block(v, w_ref, b_ref, gg_ref, bb_ref):
        y = jnp.dot(v, w_ref[...], preferred_element_type=F32) + b_ref[...]
        y = jnp.where(y >= 0.0, y, 0.15 * y)
        return gg_ref[...] * y + bb_ref[...]

    z = z_ref[...]
    z = block(z, d1_ref, db1_ref, g1_ref, bb1_ref)
    z = block(z, d2_ref, db2_ref, g2_ref, bb2_ref)
    z = block(z, d3_ref, db3_ref, g3_ref, bb3_ref)
    t = jnp.dot(z, wh_ref[...], preferred_element_type=F32) + bh_ref[...]
    lg = t[:, 0:1]
    za = jax.nn.sigmoid(t[:, 1:3])
    sg = jnp.abs(t[:, 3:5]) + 1e-5
    out_ref[...] = jnp.concatenate(
        [lg, za[:, 0:1] * math.pi, za[:, 1:2] * (2.0 * math.pi), sg], axis=1)


def _decoder(z, ws):
    (d1, db1, g1, bb1, d2, db2, g2, bb2, d3, db3, g3, bb3, wh, bh) = ws
    return pl.pallas_call(
        _decoder_body,
        out_shape=_sds((G, 5)),
    )(z, d1, db1, g1, bb1, d2, db2, g2, bb2, d3, db3, g3, bb3, wh, bh)


def kernel(x, edge_index, i, params):
    p = params
    scale = 1.0 / math.sqrt(1.0 + EPS)

    send = edge_index[:, 0].astype(jnp.int32)
    recv = edge_index[:, 1].astype(jnp.int32)
    zpad = jnp.zeros((EP - E,), jnp.int32)
    send_p = jnp.concatenate([send, zpad])
    recv_p = jnp.concatenate([recv, zpad])

    w1 = p['mp_msg_w1']
    a_recv, a_send, ew = w1[0:D], w1[D:2 * D], w1[2 * D:]
    ep_w = (p['bn_e_gamma'] * scale)[:, None] * ew
    pmat = jnp.zeros((D, 4 * H), F32).at[3:D].set(ep_w[0:D - 3])
    wr = a_recv + pmat
    ws = a_send - pmat
    wd = ep_w[D - 3:D - 2]
    wv = jnp.concatenate([ep_w[D - 2:], jnp.zeros((5, 4 * H), F32)])
    c0 = (p['bn_e_beta'] @ ew + p['mp_msg_b1'])[None]
    w2 = p['mp_msg_w2']
    b2 = p['mp_msg_b2'][None]

    u1, bu1 = p['mp_upd_w1'], p['mp_upd_b1'][None]
    u2, bu2 = p['mp_upd_w2'], p['mp_upd_b2'][None]
    zpad_w = jnp.zeros((D - H, 2 * H), F32)
    s1t = jnp.concatenate([p['sage1_w'][0:H], zpad_w])
    s1b = jnp.concatenate([p['sage1_w'][H:2 * H], zpad_w])
    s1bias = p['sage1_b'][None]
    s2t, s2b = p['sage2_w'][0:2 * H], p['sage2_w'][2 * H:4 * H]
    s2bias = p['sage2_b'][None]

    def headfold(w1h, b1h, w2h, b2h, w3h, b3h):
        wt = w1h @ w2h @ w3h
        bt = b1h @ w2h @ w3h + b2h @ w3h + b3h
        return wt, bt

    lw, lb = headfold(p['loge_w1'], p['loge_b1'], p['loge_w2'], p['loge_b2'],
                      p['loge_w3'], p['loge_b3'])
    aw, ab = headfold(p['ang_w1'], p['ang_b1'], p['ang_w2'], p['ang_b2'],
                      p['ang_w3'], p['ang_b3'])
    aw = aw @ p['angsc_w']
    ab = ab @ p['angsc_w'] + p['angsc_b']
    sw, sb = headfold(p['sig_w1'], p['sig_b1'], p['sig_w2'], p['sig_b2'],
                      p['sig_w3'], p['sig_b3'])
    wh = jnp.concatenate([lw, aw, sw, jnp.zeros((8 * H, 3), F32)], axis=1)
    bh = jnp.concatenate([lb, ab, sb, jnp.zeros((3,), F32)])[None]
    dec = (p['dec_w1'], p['dec_b1'][None], (p['bn1_g'] * scale)[None],
           p['bn1_b'][None],
           p['dec_w2'], p['dec_b2'][None], (p['bn2_g'] * scale)[None],
           p['bn2_b'][None],
           p['dec_w3'], p['dec_b3'][None], (p['bn3_g'] * scale)[None],
           p['bn3_b'][None], wh, bh)

    offs = jnp.searchsorted(i, jnp.arange(G + 1, dtype=jnp.int32)
                            ).astype(jnp.int32)

    xs, xr = _sc_gather_xs_xr(x, send_p, recv_p)
    m, rm2d, rmd2d, oh = _edge_mlp(xs, xr, recv_p[:, None], wr, ws, wd, wv,
                                   c0, w2, b2)
    rm = rm2d[:, 0]
    out_m, out_cnt = _sc_scatter_edges(m, rm, rmd2d[:, 0], oh)
    part_e = out_m.reshape(2, R, D)
    cntp = out_cnt.reshape(2, CROWS * D)[:, :R, None]
    h1, cnt8 = _update_mlp(part_e, cntp, u1, bu1, u2, bu2)
    part1 = _sc_gather_scatter(h1, send_p, rm, D).reshape(2, R, D)
    hs1 = _sage_layer(h1, part1, cnt8, s1t, s1b, s1bias)
    part2 = _sc_gather_scatter(hs1, send_p, rm, 2 * H).reshape(2, R, 2 * H)
    hs2 = _sage_layer(hs1, part2, cnt8, s2t, s2b, s2bias)
    z = _pool(offs, x, hs2).reshape(G, 1280)
    return _decoder(z, dec)


# speedup vs baseline: 2.2519x; 1.1649x over previous
import functools
import math

import jax
import jax.numpy as jnp
from jax import lax
from jax.experimental import pallas as pl
from jax.experimental.pallas import tpu as pltpu
from jax.experimental.pallas import tpu_sc as plsc

N = 10000
E = 160000
D = 128
H = 64
G = 128
EPS = 1e-3

NTILE = 32
EP = 163840
PER_TILE = EP // NTILE
CH = 128
NCH = PER_TILE // CH
R = 10240
TRASH = N
RSUB = R // 16
RCH = 160
CROWS = 128
ET = 640
NT = EP // ET
F32 = jnp.float32


def _sds(shape, dtype=F32):
    return jax.ShapeDtypeStruct(shape, dtype)


def _mesh():
    return plsc.VectorSubcoreMesh(core_axis_name="c", subcore_axis_name="s")


def _sc_gather_xs_xr(x, send2d, recv2d):
    def body(x_hbm, send_hbm, recv_hbm, xs_hbm, xr_hbm,
             idxs2d, idxr2d, s0, s1, r0, r1, sem_s0, sem_s1, sem_r0, sem_r1):
        wid = lax.axis_index("c") * 16 + lax.axis_index("s")
        base = wid * PER_TILE
        pltpu.sync_copy(send_hbm.at[pl.ds(wid * NCH, NCH)], idxs2d)
        pltpu.sync_copy(recv_hbm.at[pl.ds(wid * NCH, NCH)], idxr2d)

        def step(j, carry):
            c0 = 2 * j
            c1 = 2 * j + 1
            gs0 = pltpu.async_copy(x_hbm.at[idxs2d.at[c0]], s0, sem_s0)
            gr0 = pltpu.async_copy(x_hbm.at[idxr2d.at[c0]], r0, sem_r0)
            gs1 = pltpu.async_copy(x_hbm.at[idxs2d.at[c1]], s1, sem_s1)
            gr1 = pltpu.async_copy(x_hbm.at[idxr2d.at[c1]], r1, sem_r1)
            gs0.wait()
            pltpu.sync_copy(s0, xs_hbm.at[pl.ds(base + c0 * CH, CH)])
            gr0.wait()
            pltpu.sync_copy(r0, xr_hbm.at[pl.ds(base + c0 * CH, CH)])
            gs1.wait()
            pltpu.sync_copy(s1, xs_hbm.at[pl.ds(base + c1 * CH, CH)])
            gr1.wait()
            pltpu.sync_copy(r1, xr_hbm.at[pl.ds(base + c1 * CH, CH)])
            return carry

        lax.fori_loop(0, NCH // 2, step, 0)

    k = pl.kernel(
        body,
        out_type=(_sds((EP, D)), _sds((EP, D))),
        mesh=_mesh(),
        scratch_types=[
            pltpu.VMEM((NCH, CH), jnp.int32),
            pltpu.VMEM((NCH, CH), jnp.int32),
            pltpu.VMEM((CH, D), F32),
            pltpu.VMEM((CH, D), F32),
            pltpu.VMEM((CH, D), F32),
            pltpu.VMEM((CH, D), F32),
            pltpu.SemaphoreType.DMA,
            pltpu.SemaphoreType.DMA,
            pltpu.SemaphoreType.DMA,
            pltpu.SemaphoreType.DMA,
        ],
    )
    return k(x, send2d, recv2d)


def _zero_buf(buf, nrows, width):
    def zrow(r, c2):
        def zcol(q, c3):
            buf[r, pl.ds(q * 16, 16)] = jnp.zeros((16,), F32)
            return c3
        return lax.fori_loop(0, width // 16, zcol, c2)

    lax.fori_loop(0, nrows, zrow, 0)


def _sc_scatter_edges(m, rm2d):
    def body(m_hbm, rm_hbm, out_hbm,
             idx2d, rows0, rows1, seml0, seml1, shared):
        cid = lax.axis_index("c")
        sid = lax.axis_index("s")
        wid = cid * 16 + sid
        base = wid * PER_TILE
        pltpu.sync_copy(rm_hbm.at[pl.ds(wid * NCH, NCH)], idx2d)
        _zero_buf(rows0, CH, D)
        for q in range(RSUB // CH):
            pltpu.sync_copy(rows0, shared.at[pl.ds(sid * RSUB + q * CH, CH)])
        plsc.subcore_barrier()

        def step(j, carry):
            c0 = 2 * j
            c1 = 2 * j + 1
            l0 = pltpu.async_copy(
                m_hbm.at[pl.ds(base + c0 * CH, CH)], rows0, seml0)
            l1 = pltpu.async_copy(
                m_hbm.at[pl.ds(base + c1 * CH, CH)], rows1, seml1)
            l0.wait()
            pltpu.sync_copy(rows0, shared.at[idx2d.at[c0]], add=True)
            l1.wait()
            pltpu.sync_copy(rows1, shared.at[idx2d.at[c1]], add=True)
            return carry

        lax.fori_loop(0, NCH // 2, step, 0)
        plsc.subcore_barrier()
        for q in range(RSUB // CH):
            r0 = sid * RSUB + q * CH
            pltpu.sync_copy(shared.at[pl.ds(r0, CH)], rows0)
            pltpu.sync_copy(rows0, out_hbm.at[pl.ds(cid * R + r0, CH)])

    k = pl.kernel(
        body,
        out_type=_sds((2 * R, D)),
        mesh=_mesh(),
        scratch_types=[
            pltpu.VMEM((NCH, CH), jnp.int32),
            pltpu.VMEM((CH, D), F32),
            pltpu.VMEM((CH, D), F32),
            pltpu.SemaphoreType.DMA,
            pltpu.SemaphoreType.DMA,
            pltpu.VMEM_SHARED((R, D), F32),
        ],
    )
    return k(m, rm2d)


def _sc_gather_scatter(h, send2d, rm2d):
    def body(h_hbm, send_hbm, rm_hbm, out_hbm,
             idxs2d, idxm2d, rows0, rows1, semg0, semg1, shared):
        cid = lax.axis_index("c")
        sid = lax.axis_index("s")
        wid = cid * 16 + sid
        base = wid * PER_TILE
        pltpu.sync_copy(send_hbm.at[pl.ds(wid * NCH, NCH)], idxs2d)
        pltpu.sync_copy(rm_hbm.at[pl.ds(wid * NCH, NCH)], idxm2d)
        _zero_buf(rows0, CH, D)
        for q in range(RSUB // CH):
            pltpu.sync_copy(rows0, shared.at[pl.ds(sid * RSUB + q * CH, CH)])
        plsc.subcore_barrier()

        def step(j, carry):
            c0 = 2 * j
            c1 = 2 * j + 1
            g0 = pltpu.async_copy(h_hbm.at[idxs2d.at[c0]], rows0, semg0)
            g1 = pltpu.async_copy(h_hbm.at[idxs2d.at[c1]], rows1, semg1)
            g0.wait()
            pltpu.sync_copy(rows0, shared.at[idxm2d.at[c0]], add=True)
            g1.wait()
            pltpu.sync_copy(rows1, shared.at[idxm2d.at[c1]], add=True)
            return carry

        lax.fori_loop(0, NCH // 2, step, 0)
        plsc.subcore_barrier()
        for q in range(RSUB // CH):
            r0 = sid * RSUB + q * CH
            pltpu.sync_copy(shared.at[pl.ds(r0, CH)], rows0)
            pltpu.sync_copy(rows0, out_hbm.at[pl.ds(cid * R + r0, CH)])

    k = pl.kernel(
        body,
        out_type=_sds((2 * R, D)),
        mesh=_mesh(),
        scratch_types=[
            pltpu.VMEM((NCH, CH), jnp.int32),
            pltpu.VMEM((NCH, CH), jnp.int32),
            pltpu.VMEM((CH, D), F32),
            pltpu.VMEM((CH, D), F32),
            pltpu.SemaphoreType.DMA,
            pltpu.SemaphoreType.DMA,
            pltpu.VMEM_SHARED((R, D), F32),
        ],
    )
    return k(h, send2d, rm2d)


def _edge_mlp_body(xs_ref, xr_ref, recv_ref, wr_ref, ws_ref, wd_ref, wv_ref,
                   c0_ref, w2_ref, b2_ref, mext_ref, rm_ref):
    t = pl.program_id(0)
    xs = xs_ref[...]
    xr = xr_ref[...]
    diff = xr - xs
    lane = lax.broadcasted_iota(jnp.int32, (ET, D), 1)
    d2 = jnp.sum(jnp.where(lane < 3, diff * diff, 0.0), axis=1, keepdims=True)
    dists = jnp.sqrt(d2 + 1e-24)
    inv = jnp.where(d2 > 0, 1.0 / dists, 0.0)

    m1 = jnp.dot(xr, wr_ref[...], preferred_element_type=F32)
    m1 = m1 + jnp.dot(xs, ws_ref[...], preferred_element_type=F32)
    m1 = m1 + dists * wd_ref[...]
    for kdim in range(3):
        vk = jnp.sum(jnp.where(lane == kdim, diff, 0.0), axis=1,
                     keepdims=True) * inv
        m1 = m1 + vk * wv_ref[kdim, :][None, :]
    m1 = jnp.maximum(m1 + c0_ref[...], 0.0)
    m2 = jnp.dot(m1, w2_ref[...], preferred_element_type=F32) + b2_ref[...]
    m2 = jnp.maximum(m2, 0.0)

    s3 = jnp.sum(jnp.where(lane == 3, xs, 0.0), axis=1, keepdims=True)
    r3 = jnp.sum(jnp.where(lane == 3, xr, 0.0), axis=1, keepdims=True)
    rowid = t * ET + lax.broadcasted_iota(jnp.int32, (ET, 1), 0)
    maskb = (s3 <= r3) & (rowid < E)
    mext_ref[...] = m2 * maskb.astype(F32)
    rm_ref[...] = jnp.where(maskb, recv_ref[...], TRASH)


def _edge_mlp(xs, xr, recv2d, wr, ws, wd, wv, c0, w2, b2):
    return pl.pallas_call(
        _edge_mlp_body,
        grid=(NT,),
        in_specs=[
            pl.BlockSpec((ET, D), lambda t: (t, 0)),
            pl.BlockSpec((ET, D), lambda t: (t, 0)),
            pl.BlockSpec((ET, 1), lambda t: (t, 0)),
            pl.BlockSpec((D, 256), lambda t: (0, 0)),
            pl.BlockSpec((D, 256), lambda t: (0, 0)),
            pl.BlockSpec((1, 256), lambda t: (0, 0)),
            pl.BlockSpec((8, 256), lambda t: (0, 0)),
            pl.BlockSpec((1, 256), lambda t: (0, 0)),
            pl.BlockSpec((256, D), lambda t: (0, 0)),
            pl.BlockSpec((1, D), lambda t: (0, 0)),
        ],
        out_specs=[
            pl.BlockSpec((ET, D), lambda t: (t, 0)),
            pl.BlockSpec((ET, 1), lambda t: (t, 0)),
        ],
        out_shape=[_sds((EP, D)), _sds((EP, 1), jnp.int32)],
    )(xs, xr, recv2d, wr, ws, wd, wv, c0, w2, b2)


def _update_mlp_body(p_ref, u1_ref, bu1_ref, u2_ref, bu2_ref, h1_ref):
    agg = p_ref[0] + p_ref[1]
    a1 = jnp.maximum(
        jnp.dot(agg, u1_ref[...], preferred_element_type=F32) + bu1_ref[...],
        0.0)
    h1 = jnp.maximum(
        jnp.dot(a1, u2_ref[...], preferred_element_type=F32) + bu2_ref[...],
        0.0)
    lane = lax.broadcasted_iota(jnp.int32, (RSUB, D - H), 1)
    pad = jnp.where(lane == 0, 1.0, 0.0)
    h1_ref[...] = jnp.concatenate([h1, pad], axis=1)


def _update_mlp(partials, u1, bu1, u2, bu2):
    return pl.pallas_call(
        _update_mlp_body,
        grid=(R // RSUB,),
        in_specs=[
            pl.BlockSpec((2, RSUB, D), lambda t: (0, t, 0)),
            pl.BlockSpec((2 * H, 2 * H), lambda t: (0, 0)),
            pl.BlockSpec((1, 2 * H), lambda t: (0, 0)),
            pl.BlockSpec((2 * H, H), lambda t: (0, 0)),
            pl.BlockSpec((1, H), lambda t: (0, 0)),
        ],
        out_specs=pl.BlockSpec((RSUB, D), lambda t: (t, 0)),
        out_shape=_sds((R, D)),
    )(partials, u1, bu1, u2, bu2)


def _sage1_body(h_ref, p_ref, wt_ref, wb_ref, b_ref, out_ref, cnt_ref):
    psum = p_ref[0] + p_ref[1]
    cnt = jnp.maximum(psum[:, H:H + 1], 1.0)
    cnt_ref[...] = jnp.broadcast_to(cnt, (RSUB, 8))
    nb = psum / cnt
    h = jnp.dot(h_ref[...], wt_ref[...], preferred_element_type=F32)
    h = h + jnp.dot(nb, wb_ref[...], preferred_element_type=F32) + b_ref[...]
    h = jnp.maximum(h, 0.0)
    ss = jnp.sum(h * h, axis=1, keepdims=True)
    out_ref[...] = h * lax.rsqrt(jnp.maximum(ss, 1e-12))


def _sage1_layer(h, partials, wt, wb, b):
    return pl.pallas_call(
        _sage1_body,
        grid=(R // RSUB,),
        in_specs=[
            pl.BlockSpec((RSUB, D), lambda t: (t, 0)),
            pl.BlockSpec((2, RSUB, D), lambda t: (0, t, 0)),
            pl.BlockSpec((D, D), lambda t: (0, 0)),
            pl.BlockSpec((D, D), lambda t: (0, 0)),
            pl.BlockSpec((1, D), lambda t: (0, 0)),
        ],
        out_specs=[
            pl.BlockSpec((RSUB, D), lambda t: (t, 0)),
            pl.BlockSpec((RSUB, 8), lambda t: (t, 0)),
        ],
        out_shape=[_sds((R, D)), _sds((R, 8))],
    )(h, partials, wt, wb, b)


def _sage_body(h_ref, p_ref, cnt_ref, wt_ref, wb_ref, b_ref, out_ref):
    nb = p_ref[0] + p_ref[1]
    cnt = jnp.maximum(jnp.max(cnt_ref[...], axis=1, keepdims=True), 1.0)
    nb = nb / cnt
    h = jnp.dot(h_ref[...], wt_ref[...], preferred_element_type=F32)
    h = h + jnp.dot(nb, wb_ref[...], preferred_element_type=F32) + b_ref[...]
    h = jnp.maximum(h, 0.0)
    ss = jnp.sum(h * h, axis=1, keepdims=True)
    out_ref[...] = h * lax.rsqrt(jnp.maximum(ss, 1e-12))


def _sage_layer(h, partials, cnt8, wt, wb, b):
    win = wt.shape[0]
    wout = wt.shape[1]
    return pl.pallas_call(
        _sage_body,
        grid=(R // RSUB,),
        in_specs=[
            pl.BlockSpec((RSUB, win), lambda t: (t, 0)),
            pl.BlockSpec((2, RSUB, win), lambda t: (0, t, 0)),
            pl.BlockSpec((RSUB, 8), lambda t: (t, 0)),
            pl.BlockSpec((win, wout), lambda t: (0, 0)),
            pl.BlockSpec((win, wout), lambda t: (0, 0)),
            pl.BlockSpec((1, wout), lambda t: (0, 0)),
        ],
        out_specs=pl.BlockSpec((RSUB, wout), lambda t: (t, 0)),
        out_shape=_sds((R, wout)),
    )(h, partials, cnt8, wt, wb, b)


def _pool_body(offs_ref, x_ref, h_ref, z_ref):
    g = pl.program_id(0)
    start = offs_ref[g]
    end = offs_ref[g + 1]
    b0 = start // 8
    b1 = (end + 7) // 8
    neg = jnp.float32(-jnp.inf)
    pos = jnp.float32(jnp.inf)

    def step(b, carry):
        sx, sxx, mx, mn, sh, mh = carry
        xb = x_ref[pl.ds(b * 8, 8), :]
        hb = h_ref[pl.ds(b * 8, 8), :]
        row = b * 8 + lax.broadcasted_iota(jnp.int32, (8, 1), 0)
        rm = (row >= start) & (row < end)
        xv = jnp.where(rm, xb, 0.0)
        hv = jnp.where(rm, hb, 0.0)
        sx = sx + xv
        sxx = sxx + xv * xv
        mx = jnp.maximum(mx, jnp.where(rm, xb, neg))
        mn = jnp.minimum(mn, jnp.where(rm, xb, pos))
        sh = sh + hv
        mh = jnp.maximum(mh, jnp.where(rm, hb, neg))
        return (sx, sxx, mx, mn, sh, mh)

    init = (jnp.zeros((8, D), F32), jnp.zeros((8, D), F32),
            jnp.full((8, D), neg), jnp.full((8, D), pos),
            jnp.zeros((8, 4 * H), F32), jnp.full((8, 4 * H), neg))
    sx, sxx, mx, mn, sh, mh = lax.fori_loop(b0, b1, step, init)

    nf = (end - start).astype(F32)
    has = nf > 0.0
    cg = jnp.maximum(nf, 1.0)
    gsum = jnp.sum(sx, axis=0, keepdims=True)
    gss = jnp.sum(sxx, axis=0, keepdims=True)
    gmx = jnp.where(has, jnp.max(mx, axis=0, keepdims=True), 0.0)
    gmn = jnp.where(has, jnp.min(mn, axis=0, keepdims=True), 0.0)
    hsum = jnp.sum(sh, axis=0, keepdims=True)
    hmx = jnp.where(has, jnp.max(mh, axis=0, keepdims=True), 0.0)
    gmean = gsum / cg
    gvar = jnp.abs(gss / cg - gmean * gmean)
    pavg = hsum / cg
    z_ref[0] = jnp.concatenate(
        [hmx, pavg, hsum, gmean, gvar, gmx, gmn], axis=1)


def _pool(offs, x, h2):
    grid_spec = pltpu.PrefetchScalarGridSpec(
        num_scalar_prefetch=1,
        grid=(G,),
        in_specs=[
            pl.BlockSpec((N, D), lambda g, offs: (0, 0)),
            pl.BlockSpec((R, 4 * H), lambda g, offs: (0, 0)),
        ],
        out_specs=pl.BlockSpec((1, 1, 1280), lambda g, offs: (g, 0, 0)),
    )
    return pl.pallas_call(
        _pool_body,
        grid_spec=grid_spec,
        out_shape=_sds((G, 1, 1280)),
    )(offs, x, h2)


def _decoder_body(z_ref, d1_ref, db1_ref, g1_ref, bb1_ref,
                  d2_ref, db2_ref, g2_ref, bb2_ref,
                  d3_ref, db3_ref, g3_ref, bb3_ref,
                  wh_ref, bh_ref, out_ref):
    def block(v, w_ref, b_ref, gg_ref, bb_ref):
        y = jnp.dot(v, w_ref[...], preferred_element_type=F32) + b_ref[...]
        y = jnp.where(y >= 0.0, y, 0.15 * y)
        return gg_ref[...] * y + bb_ref[...]

    z = z_ref[...]
    z = block(z, d1_ref, db1_ref, g1_ref, bb1_ref)
    z = block(z, d2_ref, db2_ref, g2_ref, bb2_ref)
    z = block(z, d3_ref, db3_ref, g3_ref, bb3_ref)
    t = jnp.dot(z, wh_ref[...], preferred_element_type=F32) + bh_ref[...]
    lg = t[:, 0:1]
    za = jax.nn.sigmoid(t[:, 1:3])
    sg = jnp.abs(t[:, 3:5]) + 1e-5
    out_ref[...] = jnp.concatenate(
        [lg, za[:, 0:1] * math.pi, za[:, 1:2] * (2.0 * math.pi), sg], axis=1)


def _decoder(z, ws):
    (d1, db1, g1, bb1, d2, db2, g2, bb2, d3, db3, g3, bb3, wh, bh) = ws
    return pl.pallas_call(
        _decoder_body,
        out_shape=_sds((G, 5)),
    )(z, d1, db1, g1, bb1, d2, db2, g2, bb2, d3, db3, g3, bb3, wh, bh)


def kernel(x, edge_index, i, params):
    p = params
    scale = 1.0 / math.sqrt(1.0 + EPS)

    send = edge_index[:, 0].astype(jnp.int32)
    recv = edge_index[:, 1].astype(jnp.int32)
    zpad = jnp.zeros((EP - E,), jnp.int32)
    send_p = jnp.concatenate([send, zpad])
    recv_p = jnp.concatenate([recv, zpad])

    w1 = p['mp_msg_w1']
    a_recv, a_send, ew = w1[0:D], w1[D:2 * D], w1[2 * D:]
    ep_w = (p['bn_e_gamma'] * scale)[:, None] * ew
    pmat = jnp.zeros((D, 4 * H), F32).at[3:D].set(ep_w[0:D - 3])
    wr = a_recv + pmat
    ws = a_send - pmat
    wd = ep_w[D - 3:D - 2]
    wv = jnp.concatenate([ep_w[D - 2:], jnp.zeros((5, 4 * H), F32)])
    c0 = (p['bn_e_beta'] @ ew + p['mp_msg_b1'])[None]
    w2 = p['mp_msg_w2']
    b2 = p['mp_msg_b2'][None]

    u1, bu1 = p['mp_upd_w1'], p['mp_upd_b1'][None]
    u2, bu2 = p['mp_upd_w2'], p['mp_upd_b2'][None]
    zpad_w = jnp.zeros((D - H, 2 * H), F32)
    s1t = jnp.concatenate([p['sage1_w'][0:H], zpad_w])
    s1b = jnp.concatenate([p['sage1_w'][H:2 * H], zpad_w])
    s1bias = p['sage1_b'][None]
    s2t, s2b = p['sage2_w'][0:2 * H], p['sage2_w'][2 * H:4 * H]
    s2bias = p['sage2_b'][None]

    def headfold(w1h, b1h, w2h, b2h, w3h, b3h):
        wt = w1h @ w2h @ w3h
        bt = b1h @ w2h @ w3h + b2h @ w3h + b3h
        return wt, bt

    lw, lb = headfold(p['loge_w1'], p['loge_b1'], p['loge_w2'], p['loge_b2'],
                      p['loge_w3'], p['loge_b3'])
    aw, ab = headfold(p['ang_w1'], p['ang_b1'], p['ang_w2'], p['ang_b2'],
                      p['ang_w3'], p['ang_b3'])
    aw = aw @ p['angsc_w']
    ab = ab @ p['angsc_w'] + p['angsc_b']
    sw, sb = headfold(p['sig_w1'], p['sig_b1'], p['sig_w2'], p['sig_b2'],
                      p['sig_w3'], p['sig_b3'])
    wh = jnp.concatenate([lw, aw, sw, jnp.zeros((8 * H, 3), F32)], axis=1)
    bh = jnp.concatenate([lb, ab, sb, jnp.zeros((3,), F32)])[None]
    dec = (p['dec_w1'], p['dec_b1'][None], (p['bn1_g'] * scale)[None],
           p['bn1_b'][None],
           p['dec_w2'], p['dec_b2'][None], (p['bn2_g'] * scale)[None],
           p['bn2_b'][None],
           p['dec_w3'], p['dec_b3'][None], (p['bn3_g'] * scale)[None],
           p['bn3_b'][None], wh, bh)

    offs = jnp.searchsorted(i, jnp.arange(G + 1, dtype=jnp.int32)
                            ).astype(jnp.int32)

    send2d = send_p.reshape(EP // CH, CH)
    recv2d = recv_p.reshape(EP // CH, CH)
    xs, xr = _sc_gather_xs_xr(x, send2d, recv2d)
    m, rmc = _edge_mlp(xs, xr, recv_p[:, None], wr, ws, wd, wv, c0, w2, b2)
    rm2d = rmc.reshape(EP // CH, CH)
    part_e = _sc_scatter_edges(m, rm2d).reshape(2, R, D)
    h1 = _update_mlp(part_e, u1, bu1, u2, bu2)
    part1 = _sc_gather_scatter(h1, send2d, rm2d).reshape(2, R, D)
    hs1, cnt8 = _sage1_layer(h1, part1, s1t, s1b, s1bias)
    part2 = _sc_gather_scatter(hs1, send2d, rm2d).reshape(2, R, D)
    hs2 = _sage_layer(hs1, part2, cnt8, s2t, s2b, s2bias)
    z = _pool(offs, x, hs2).reshape(G, 1280)
    return _decoder(z, dec)


# speedup vs baseline: 2.2618x; 1.0044x over previous
import functools
import math

import jax
import jax.numpy as jnp
from jax import lax
from jax.experimental import pallas as pl
from jax.experimental.pallas import tpu as pltpu
from jax.experimental.pallas import tpu_sc as plsc

N = 10000
E = 160000
D = 128
H = 64
G = 128
EPS = 1e-3

NTILE = 32
EP = 163840
PER_TILE = EP // NTILE
CH = 128
NCH = PER_TILE // CH
R = 10240
TRASH = N
RSUB = R // 16
RCH = 160
CROWS = 128
ET = 640
NT = EP // ET
F32 = jnp.float32


def _sds(shape, dtype=F32):
    return jax.ShapeDtypeStruct(shape, dtype)


def _mesh():
    return plsc.VectorSubcoreMesh(core_axis_name="c", subcore_axis_name="s")


def _sc_gather_xs_xr(x, send2d, recv2d):
    def body(x_hbm, send_hbm, recv_hbm, xs_hbm, xr_hbm,
             idxs2d, idxr2d, s0, s1, r0, r1, sem_s0, sem_s1, sem_r0, sem_r1,
             sem_w0, sem_w1, sem_w2, sem_w3):
        wid = lax.axis_index("c") * 16 + lax.axis_index("s")
        base = wid * PER_TILE
        pltpu.sync_copy(send_hbm.at[pl.ds(wid * NCH, NCH)], idxs2d)
        pltpu.sync_copy(recv_hbm.at[pl.ds(wid * NCH, NCH)], idxr2d)

        def step(j, carry):
            c0 = 2 * j
            c1 = 2 * j + 1
            gs0 = pltpu.async_copy(x_hbm.at[idxs2d.at[c0]], s0, sem_s0)
            gr0 = pltpu.async_copy(x_hbm.at[idxr2d.at[c0]], r0, sem_r0)
            gs1 = pltpu.async_copy(x_hbm.at[idxs2d.at[c1]], s1, sem_s1)
            gr1 = pltpu.async_copy(x_hbm.at[idxr2d.at[c1]], r1, sem_r1)
            gs0.wait()
            ws0 = pltpu.async_copy(
                s0, xs_hbm.at[pl.ds(base + c0 * CH, CH)], sem_w0)
            gr0.wait()
            wr0 = pltpu.async_copy(
                r0, xr_hbm.at[pl.ds(base + c0 * CH, CH)], sem_w1)
            gs1.wait()
            ws1 = pltpu.async_copy(
                s1, xs_hbm.at[pl.ds(base + c1 * CH, CH)], sem_w2)
            gr1.wait()
            wr1 = pltpu.async_copy(
                r1, xr_hbm.at[pl.ds(base + c1 * CH, CH)], sem_w3)
            ws0.wait()
            wr0.wait()
            ws1.wait()
            wr1.wait()
            return carry

        lax.fori_loop(0, NCH // 2, step, 0)

    k = pl.kernel(
        body,
        out_type=(_sds((EP, D)), _sds((EP, D))),
        mesh=_mesh(),
        scratch_types=[
            pltpu.VMEM((NCH, CH), jnp.int32),
            pltpu.VMEM((NCH, CH), jnp.int32),
            pltpu.VMEM((CH, D), F32),
            pltpu.VMEM((CH, D), F32),
            pltpu.VMEM((CH, D), F32),
            pltpu.VMEM((CH, D), F32),
            pltpu.SemaphoreType.DMA,
            pltpu.SemaphoreType.DMA,
            pltpu.SemaphoreType.DMA,
            pltpu.SemaphoreType.DMA,
            pltpu.SemaphoreType.DMA,
            pltpu.SemaphoreType.DMA,
            pltpu.SemaphoreType.DMA,
            pltpu.SemaphoreType.DMA,
        ],
    )
    return k(x, send2d, recv2d)


def _zero_buf(buf, nrows, width):
    def zrow(r, c2):
        def zcol(q, c3):
            buf[r, pl.ds(q * 16, 16)] = jnp.zeros((16,), F32)
            return c3
        return lax.fori_loop(0, width // 16, zcol, c2)

    lax.fori_loop(0, nrows, zrow, 0)


def _sc_scatter_edges(m, rm2d):
    def body(m_hbm, rm_hbm, out_hbm,
             idx2d, rows0, rows1, seml0, seml1, semw0, semw1, shared):
        cid = lax.axis_index("c")
        sid = lax.axis_index("s")
        wid = cid * 16 + sid
        base = wid * PER_TILE
        pltpu.sync_copy(rm_hbm.at[pl.ds(wid * NCH, NCH)], idx2d)
        _zero_buf(rows0, CH, D)
        for q in range(RSUB // CH):
            pltpu.sync_copy(rows0, shared.at[pl.ds(sid * RSUB + q * CH, CH)])
        plsc.subcore_barrier()

        def step(j, carry):
            c0 = 2 * j
            c1 = 2 * j + 1
            l0 = pltpu.async_copy(
                m_hbm.at[pl.ds(base + c0 * CH, CH)], rows0, seml0)
            l1 = pltpu.async_copy(
                m_hbm.at[pl.ds(base + c1 * CH, CH)], rows1, seml1)
            l0.wait()
            a0 = pltpu.async_copy(rows0, shared.at[idx2d.at[c0]], semw0,
                                  add=True)
            l1.wait()
            a1 = pltpu.async_copy(rows1, shared.at[idx2d.at[c1]], semw1,
                                  add=True)
            a0.wait()
            a1.wait()
            return carry

        lax.fori_loop(0, NCH // 2, step, 0)
        plsc.subcore_barrier()
        for q in range(RSUB // CH):
            r0 = sid * RSUB + q * CH
            pltpu.sync_copy(shared.at[pl.ds(r0, CH)], rows0)
            pltpu.sync_copy(rows0, out_hbm.at[pl.ds(cid * R + r0, CH)])

    k = pl.kernel(
        body,
        out_type=_sds((2 * R, D)),
        mesh=_mesh(),
        scratch_types=[
            pltpu.VMEM((NCH, CH), jnp.int32),
            pltpu.VMEM((CH, D), F32),
            pltpu.VMEM((CH, D), F32),
            pltpu.SemaphoreType.DMA,
            pltpu.SemaphoreType.DMA,
            pltpu.SemaphoreType.DMA,
            pltpu.SemaphoreType.DMA,
            pltpu.VMEM_SHARED((R, D), F32),
        ],
    )
    return k(m, rm2d)


def _sc_gather_scatter(h, send2d, rm2d):
    def body(h_hbm, send_hbm, rm_hbm, out_hbm,
             idxs2d, idxm2d, rows0, rows1, semg0, semg1, semw0, semw1,
             shared):
        cid = lax.axis_index("c")
        sid = lax.axis_index("s")
        wid = cid * 16 + sid
        base = wid * PER_TILE
        pltpu.sync_copy(send_hbm.at[pl.ds(wid * NCH, NCH)], idxs2d)
        pltpu.sync_copy(rm_hbm.at[pl.ds(wid * NCH, NCH)], idxm2d)
        _zero_buf(rows0, CH, D)
        for q in range(RSUB // CH):
            pltpu.sync_copy(rows0, shared.at[pl.ds(sid * RSUB + q * CH, CH)])
        plsc.subcore_barrier()

        def step(j, carry):
            c0 = 2 * j
            c1 = 2 * j + 1
            g0 = pltpu.async_copy(h_hbm.at[idxs2d.at[c0]], rows0, semg0)
            g1 = pltpu.async_copy(h_hbm.at[idxs2d.at[c1]], rows1, semg1)
            g0.wait()
            a0 = pltpu.async_copy(rows0, shared.at[idxm2d.at[c0]], semw0,
                                  add=True)
            g1.wait()
            a1 = pltpu.async_copy(rows1, shared.at[idxm2d.at[c1]], semw1,
                                  add=True)
            a0.wait()
            a1.wait()
            return carry

        lax.fori_loop(0, NCH // 2, step, 0)
        plsc.subcore_barrier()
        for q in range(RSUB // CH):
            r0 = sid * RSUB + q * CH
            pltpu.sync_copy(shared.at[pl.ds(r0, CH)], rows0)
            pltpu.sync_copy(rows0, out_hbm.at[pl.ds(cid * R + r0, CH)])

    k = pl.kernel(
        body,
        out_type=_sds((2 * R, D)),
        mesh=_mesh(),
        scratch_types=[
            pltpu.VMEM((NCH, CH), jnp.int32),
            pltpu.VMEM((NCH, CH), jnp.int32),
            pltpu.VMEM((CH, D), F32),
            pltpu.VMEM((CH, D), F32),
            pltpu.SemaphoreType.DMA,
            pltpu.SemaphoreType.DMA,
            pltpu.SemaphoreType.DMA,
            pltpu.SemaphoreType.DMA,
            pltpu.VMEM_SHARED((R, D), F32),
        ],
    )
    return k(h, send2d, rm2d)


def _edge_mlp_body(xs_ref, xr_ref, recv_ref, wr_ref, ws_ref, wd_ref, wv_ref,
                   c0_ref, w2_ref, b2_ref, mext_ref, rm_ref):
    t = pl.program_id(0)
    xs = xs_ref[...]
    xr = xr_ref[...]
    diff = xr - xs
    lane = lax.broadcasted_iota(jnp.int32, (ET, D), 1)
    d2 = jnp.sum(jnp.where(lane < 3, diff * diff, 0.0), axis=1, keepdims=True)
    dists = jnp.sqrt(d2 + 1e-24)
    inv = jnp.where(d2 > 0, 1.0 / dists, 0.0)

    m1 = jnp.dot(xr, wr_ref[...], preferred_element_type=F32)
    m1 = m1 + jnp.dot(xs, ws_ref[...], preferred_element_type=F32)
    m1 = m1 + dists * wd_ref[...]
    for kdim in range(3):
        vk = jnp.sum(jnp.where(lane == kdim, diff, 0.0), axis=1,
                     keepdims=True) * inv
        m1 = m1 + vk * wv_ref[kdim, :][None, :]
    m1 = jnp.maximum(m1 + c0_ref[...], 0.0)
    m2 = jnp.dot(m1, w2_ref[...], preferred_element_type=F32) + b2_ref[...]
    m2 = jnp.maximum(m2, 0.0)

    s3 = jnp.sum(jnp.where(lane == 3, xs, 0.0), axis=1, keepdims=True)
    r3 = jnp.sum(jnp.where(lane == 3, xr, 0.0), axis=1, keepdims=True)
    rowid = t * ET + lax.broadcasted_iota(jnp.int32, (ET, 1), 0)
    maskb = (s3 <= r3) & (rowid < E)
    mext_ref[...] = m2 * maskb.astype(F32)
    rm_ref[...] = jnp.where(maskb, recv_ref[...], TRASH)


def _edge_mlp(xs, xr, recv2d, wr, ws, wd, wv, c0, w2, b2):
    return pl.pallas_call(
        _edge_mlp_body,
        grid=(NT,),
        in_specs=[
            pl.BlockSpec((ET, D), lambda t: (t, 0)),
            pl.BlockSpec((ET, D), lambda t: (t, 0)),
            pl.BlockSpec((ET, 1), lambda t: (t, 0)),
            pl.BlockSpec((D, 256), lambda t: (0, 0)),
            pl.BlockSpec((D, 256), lambda t: (0, 0)),
            pl.BlockSpec((1, 256), lambda t: (0, 0)),
            pl.BlockSpec((8, 256), lambda t: (0, 0)),
            pl.BlockSpec((1, 256), lambda t: (0, 0)),
            pl.BlockSpec((256, D), lambda t: (0, 0)),
            pl.BlockSpec((1, D), lambda t: (0, 0)),
        ],
        out_specs=[
            pl.BlockSpec((ET, D), lambda t: (t, 0)),
            pl.BlockSpec((ET, 1), lambda t: (t, 0)),
        ],
        out_shape=[_sds((EP, D)), _sds((EP, 1), jnp.int32)],
    )(xs, xr, recv2d, wr, ws, wd, wv, c0, w2, b2)


def _update_mlp_body(p_ref, u1_ref, bu1_ref, u2_ref, bu2_ref, h1_ref):
    agg = p_ref[0] + p_ref[1]
    a1 = jnp.maximum(
        jnp.dot(agg, u1_ref[...], preferred_element_type=F32) + bu1_ref[...],
        0.0)
    h1 = jnp.maximum(
        jnp.dot(a1, u2_ref[...], preferred_element_type=F32) + bu2_ref[...],
        0.0)
    lane = lax.broadcasted_iota(jnp.int32, (RSUB, D - H), 1)
    pad = jnp.where(lane == 0, 1.0, 0.0)
    h1_ref[...] = jnp.concatenate([h1, pad], axis=1)


def _update_mlp(partials, u1, bu1, u2, bu2):
    return pl.pallas_call(
        _update_mlp_body,
        grid=(R // RSUB,),
        in_specs=[
            pl.BlockSpec((2, RSUB, D), lambda t: (0, t, 0)),
            pl.BlockSpec((2 * H, 2 * H), lambda t: (0, 0)),
            pl.BlockSpec((1, 2 * H), lambda t: (0, 0)),
            pl.BlockSpec((2 * H, H), lambda t: (0, 0)),
            pl.BlockSpec((1, H), lambda t: (0, 0)),
        ],
        out_specs=pl.BlockSpec((RSUB, D), lambda t: (t, 0)),
        out_shape=_sds((R, D)),
    )(partials, u1, bu1, u2, bu2)


def _sage1_body(h_ref, p_ref, wt_ref, wb_ref, b_ref, out_ref, cnt_ref):
    psum = p_ref[0] + p_ref[1]
    cnt = jnp.maximum(psum[:, H:H + 1], 1.0)
    cnt_ref[...] = jnp.broadcast_to(cnt, (RSUB, 8))
    nb = psum / cnt
    h = jnp.dot(h_ref[...], wt_ref[...], preferred_element_type=F32)
    h = h + jnp.dot(nb, wb_ref[...], preferred_element_type=F32) + b_ref[...]
    h = jnp.maximum(h, 0.0)
    ss = jnp.sum(h * h, axis=1, keepdims=True)
    out_ref[...] = h * lax.rsqrt(jnp.maximum(ss, 1e-12))


def _sage1_layer(h, partials, wt, wb, b):
    return pl.pallas_call(
        _sage1_body,
        grid=(R // RSUB,),
        in_specs=[
            pl.BlockSpec((RSUB, D), lambda t: (t, 0)),
            pl.BlockSpec((2, RSUB, D), lambda t: (0, t, 0)),
            pl.BlockSpec((D, D), lambda t: (0, 0)),
            pl.BlockSpec((D, D), lambda t: (0, 0)),
            pl.BlockSpec((1, D), lambda t: (0, 0)),
        ],
        out_specs=[
            pl.BlockSpec((RSUB, D), lambda t: (t, 0)),
            pl.BlockSpec((RSUB, 8), lambda t: (t, 0)),
        ],
        out_shape=[_sds((R, D)), _sds((R, 8))],
    )(h, partials, wt, wb, b)


def _sage_body(h_ref, p_ref, cnt_ref, wt_ref, wb_ref, b_ref, out_ref):
    nb = p_ref[0] + p_ref[1]
    cnt = jnp.maximum(jnp.max(cnt_ref[...], axis=1, keepdims=True), 1.0)
    nb = nb / cnt
    h = jnp.dot(h_ref[...], wt_ref[...], preferred_element_type=F32)
    h = h + jnp.dot(nb, wb_ref[...], preferred_element_type=F32) + b_ref[...]
    h = jnp.maximum(h, 0.0)
    ss = jnp.sum(h * h, axis=1, keepdims=True)
    out_ref[...] = h * lax.rsqrt(jnp.maximum(ss, 1e-12))


def _sage_layer(h, partials, cnt8, wt, wb, b):
    win = wt.shape[0]
    wout = wt.shape[1]
    return pl.pallas_call(
        _sage_body,
        grid=(R // RSUB,),
        in_specs=[
            pl.BlockSpec((RSUB, win), lambda t: (t, 0)),
            pl.BlockSpec((2, RSUB, win), lambda t: (0, t, 0)),
            pl.BlockSpec((RSUB, 8), lambda t: (t, 0)),
            pl.BlockSpec((win, wout), lambda t: (0, 0)),
            pl.BlockSpec((win, wout), lambda t: (0, 0)),
            pl.BlockSpec((1, wout), lambda t: (0, 0)),
        ],
        out_specs=pl.BlockSpec((RSUB, wout), lambda t: (t, 0)),
        out_shape=_sds((R, wout)),
    )(h, partials, cnt8, wt, wb, b)


def _pool_body(offs_ref, x_ref, h_ref, z_ref):
    g = pl.program_id(0)
    start = offs_ref[g]
    end = offs_ref[g + 1]
    b0 = start // 8
    b1 = (end + 7) // 8
    neg = jnp.float32(-jnp.inf)
    pos = jnp.float32(jnp.inf)

    def step(b, carry):
        sx, sxx, mx, mn, sh, mh = carry
        xb = x_ref[pl.ds(b * 8, 8), :]
        hb = h_ref[pl.ds(b * 8, 8), :]
        row = b * 8 + lax.broadcasted_iota(jnp.int32, (8, 1), 0)
        rm = (row >= start) & (row < end)
        xv = jnp.where(rm, xb, 0.0)
        hv = jnp.where(rm, hb, 0.0)
        sx = sx + xv
        sxx = sxx + xv * xv
        mx = jnp.maximum(mx, jnp.where(rm, xb, neg))
        mn = jnp.minimum(mn, jnp.where(rm, xb, pos))
        sh = sh + hv
        mh = jnp.maximum(mh, jnp.where(rm, hb, neg))
        return (sx, sxx, mx, mn, sh, mh)

    init = (jnp.zeros((8, D), F32), jnp.zeros((8, D), F32),
            jnp.full((8, D), neg), jnp.full((8, D), pos),
            jnp.zeros((8, 4 * H), F32), jnp.full((8, 4 * H), neg))
    sx, sxx, mx, mn, sh, mh = lax.fori_loop(b0, b1, step, init)

    nf = (end - start).astype(F32)
    has = nf > 0.0
    cg = jnp.maximum(nf, 1.0)
    gsum = jnp.sum(sx, axis=0, keepdims=True)
    gss = jnp.sum(sxx, axis=0, keepdims=True)
    gmx = jnp.where(has, jnp.max(mx, axis=0, keepdims=True), 0.0)
    gmn = jnp.where(has, jnp.min(mn, axis=0, keepdims=True), 0.0)
    hsum = jnp.sum(sh, axis=0, keepdims=True)
    hmx = jnp.where(has, jnp.max(mh, axis=0, keepdims=True), 0.0)
    gmean = gsum / cg
    gvar = jnp.abs(gss / cg - gmean * gmean)
    pavg = hsum / cg
    z_ref[0] = jnp.concatenate(
        [hmx, pavg, hsum, gmean, gvar, gmx, gmn], axis=1)


def _pool(offs, x, h2):
    grid_spec = pltpu.PrefetchScalarGridSpec(
        num_scalar_prefetch=1,
        grid=(G,),
        in_specs=[
            pl.BlockSpec((N, D), lambda g, offs: (0, 0)),
            pl.BlockSpec((R, 4 * H), lambda g, offs: (0, 0)),
        ],
        out_specs=pl.BlockSpec((1, 1, 1280), lambda g, offs: (g, 0, 0)),
    )
    return pl.pallas_call(
        _pool_body,
        grid_spec=grid_spec,
        out_shape=_sds((G, 1, 1280)),
    )(offs, x, h2)


def _decoder_body(z_ref, d1_ref, db1_ref, g1_ref, bb1_ref,
                  d2_ref, db2_ref, g2_ref, bb2_ref,
                  d3_ref, db3_ref, g3_ref, bb3_ref,
                  wh_ref, bh_ref, out_ref):
    def block(v, w_ref, b_ref, gg_ref, bb_ref):
        y = jnp.dot(v, w_ref[...], preferred_element_type=F32) + b_ref[...]
        y = jnp.where(y >= 0.0, y, 0.15 * y)
        return gg_ref[...] * y + bb_ref[...]

    z = z_ref[...]
    z = block(z, d1_ref, db1_ref, g1_ref, bb1_ref)
    z = block(z, d2_ref, db2_ref, g2_ref, bb2_ref)
    z = block(z, d3_ref, db3_ref, g3_ref, bb3_ref)
    t = jnp.dot(z, wh_ref[...], preferred_element_type=F32) + bh_ref[...]
    lg = t[:, 0:1]
    za = jax.nn.sigmoid(t[:, 1:3])
    sg = jnp.abs(t[:, 3:5]) + 1e-5
    out_ref[...] = jnp.concatenate(
        [lg, za[:, 0:1] * math.pi, za[:, 1:2] * (2.0 * math.pi), sg], axis=1)


def _decoder(z, ws):
    (d1, db1, g1, bb1, d2, db2, g2, bb2, d3, db3, g3, bb3, wh, bh) = ws
    return pl.pallas_call(
        _decoder_body,
        out_shape=_sds((G, 5)),
    )(z, d1, db1, g1, bb1, d2, db2, g2, bb2, d3, db3, g3, bb3, wh, bh)


def kernel(x, edge_index, i, params):
    p = params
    scale = 1.0 / math.sqrt(1.0 + EPS)

    send = edge_index[:, 0].astype(jnp.int32)
    recv = edge_index[:, 1].astype(jnp.int32)
    zpad = jnp.zeros((EP - E,), jnp.int32)
    send_p = jnp.concatenate([send, zpad])
    recv_p = jnp.concatenate([recv, zpad])

    w1 = p['mp_msg_w1']
    a_recv, a_send, ew = w1[0:D], w1[D:2 * D], w1[2 * D:]
    ep_w = (p['bn_e_gamma'] * scale)[:, None] * ew
    pmat = jnp.zeros((D, 4 * H), F32).at[3:D].set(ep_w[0:D - 3])
    wr = a_recv + pmat
    ws = a_send - pmat
    wd = ep_w[D - 3:D - 2]
    wv = jnp.concatenate([ep_w[D - 2:], jnp.zeros((5, 4 * H), F32)])
    c0 = (p['bn_e_beta'] @ ew + p['mp_msg_b1'])[None]
    w2 = p['mp_msg_w2']
    b2 = p['mp_msg_b2'][None]

    u1, bu1 = p['mp_upd_w1'], p['mp_upd_b1'][None]
    u2, bu2 = p['mp_upd_w2'], p['mp_upd_b2'][None]
    zpad_w = jnp.zeros((D - H, 2 * H), F32)
    s1t = jnp.concatenate([p['sage1_w'][0:H], zpad_w])
    s1b = jnp.concatenate([p['sage1_w'][H:2 * H], zpad_w])
    s1bias = p['sage1_b'][None]
    s2t, s2b = p['sage2_w'][0:2 * H], p['sage2_w'][2 * H:4 * H]
    s2bias = p['sage2_b'][None]

    def headfold(w1h, b1h, w2h, b2h, w3h, b3h):
        wt = w1h @ w2h @ w3h
        bt = b1h @ w2h @ w3h + b2h @ w3h + b3h
        return wt, bt

    lw, lb = headfold(p['loge_w1'], p['loge_b1'], p['loge_w2'], p['loge_b2'],
                      p['loge_w3'], p['loge_b3'])
    aw, ab = headfold(p['ang_w1'], p['ang_b1'], p['ang_w2'], p['ang_b2'],
                      p['ang_w3'], p['ang_b3'])
    aw = aw @ p['angsc_w']
    ab = ab @ p['angsc_w'] + p['angsc_b']
    sw, sb = headfold(p['sig_w1'], p['sig_b1'], p['sig_w2'], p['sig_b2'],
                      p['sig_w3'], p['sig_b3'])
    wh = jnp.concatenate([lw, aw, sw, jnp.zeros((8 * H, 3), F32)], axis=1)
    bh = jnp.concatenate([lb, ab, sb, jnp.zeros((3,), F32)])[None]
    dec = (p['dec_w1'], p['dec_b1'][None], (p['bn1_g'] * scale)[None],
           p['bn1_b'][None],
           p['dec_w2'], p['dec_b2'][None], (p['bn2_g'] * scale)[None],
           p['bn2_b'][None],
           p['dec_w3'], p['dec_b3'][None], (p['bn3_g'] * scale)[None],
           p['bn3_b'][None], wh, bh)

    offs = jnp.searchsorted(i, jnp.arange(G + 1, dtype=jnp.int32)
                            ).astype(jnp.int32)

    send2d = send_p.reshape(EP // CH, CH)
    recv2d = recv_p.reshape(EP // CH, CH)
    xs, xr = _sc_gather_xs_xr(x, send2d, recv2d)
    m, rmc = _edge_mlp(xs, xr, recv_p[:, None], wr, ws, wd, wv, c0, w2, b2)
    rm2d = rmc.reshape(EP // CH, CH)
    part_e = _sc_scatter_edges(m, rm2d).reshape(2, R, D)
    h1 = _update_mlp(part_e, u1, bu1, u2, bu2)
    part1 = _sc_gather_scatter(h1, send2d, rm2d).reshape(2, R, D)
    hs1, cnt8 = _sage1_layer(h1, part1, s1t, s1b, s1bias)
    part2 = _sc_gather_scatter(hs1, send2d, rm2d).reshape(2, R, D)
    hs2 = _sage_layer(hs1, part2, cnt8, s2t, s2b, s2bias)
    z = _pool(offs, x, hs2).reshape(G, 1280)
    return _decoder(z, dec)
